# baseline (speedup 1.0000x reference)
import functools

import jax
import jax.numpy as jnp
from jax import lax
from jax.experimental import pallas as pl
from jax.experimental.pallas import tpu as pltpu
from jax.experimental.pallas import tpu_sc as plsc

NC = 2
NS = 16
L = 16
BLK = 64


def _tc_body(x_ref, w_ref, m_ref, h_ref, a_ref):
    p = lax.dot_general(
        x_ref[...], w_ref[0],
        dimension_numbers=(((1,), (1,)), ((), ())),
        preferred_element_type=jnp.float32,
    )
    h_ref[0] = p
    al = jnp.dot(p, m_ref[0], preferred_element_type=jnp.float32)
    au = lax.bitcast_convert_type(al, jnp.uint32) + jnp.uint32(0x8000)
    lo_s = au[:, 0:1] >> 16
    hi_s = au[:, 1:2] & jnp.uint32(0xFFFF0000)
    lo_t = au[:, 2:3] >> 16
    hi_t = au[:, 3:4] & jnp.uint32(0xFFFF0000)
    packed = jnp.concatenate([lo_s | hi_s, lo_t | hi_t], axis=1)
    a_ref[0] = lax.bitcast_convert_type(packed, jnp.int32)


def _make_sc_kernel(N_pad, CPW, HO2):
    NV = HO2 // L
    mesh = plsc.VectorSubcoreMesh(core_axis_name="c", subcore_axis_name="s")
    rows_per_tile = N_pad // NS
    n_fin = rows_per_tile // L
    n_blk = CPW // BLK

    @functools.partial(
        pl.kernel,
        out_type=jax.ShapeDtypeStruct((NC, N_pad, HO2), jnp.float32),
        mesh=mesh,
        scratch_types=[
            pltpu.VMEM((N_pad // 64, 128), jnp.int32),
            pltpu.VMEM((BLK // 8, 128), jnp.int32),
            pltpu.VMEM((BLK // 8, 128), jnp.int32),
            pltpu.VMEM((2 * L,), jnp.int32),
            pltpu.VMEM((2 * L,), jnp.int32),
            pltpu.VMEM((2 * L,), jnp.int32),
            pltpu.VMEM((2 * L,), jnp.int32),
            pltpu.VMEM((2 * L,), jnp.int32),
            pltpu.VMEM((2 * L,), jnp.int32),
            pltpu.VMEM((2 * L, HO2), jnp.float32),
            pltpu.VMEM((2 * L, HO2), jnp.float32),
            pltpu.VMEM((2 * L, 128), jnp.float32),
            pltpu.VMEM_SHARED((N_pad, HO2), jnp.float32),
            pltpu.VMEM_SHARED((N_pad * L // 128, 128), jnp.float32),
        ] + [pltpu.SemaphoreType.DMA] * 7,
        compiler_params=pltpu.CompilerParams(needs_layout_passes=False),
    )
    def sc_fn(h2, alpha_h, srch, tgth, out, alpha_v, srcb, tgtb,
              ixg0, ixg1, ixs0, ixs1, ixd0, ixd1, rows,
              rows1, wden, acc, den,
              semg0, semg1, semg2, semg3, semsr, semd0, semd1):
        c = lax.axis_index("c")
        s = lax.axis_index("s")
        zvec = jnp.zeros((L,), jnp.float32)
        iota = lax.iota(jnp.int32, L)

        pltpu.sync_copy(alpha_h.at[c], alpha_v)

        for r in range(2 * L):
            for q in range(8):
                rows[r, pl.ds(q * L, L)] = zvec
                wden[r, pl.ds(q * L, L)] = zvec
        zbase = s * rows_per_tile

        def zero_body(k, carry):
            pltpu.sync_copy(rows, acc.at[pl.ds(zbase + k * 2 * L, 2 * L)])
            return carry

        lax.fori_loop(0, n_fin // 2, zero_body, 0)
        dbase = s * (rows_per_tile // 8)
        dleft = rows_per_tile // 8
        doff = 0
        while dleft >= 2 * L:
            pltpu.sync_copy(wden, den.at[pl.ds(dbase + doff, 2 * L)])
            doff += 2 * L
            dleft -= 2 * L
        while dleft >= L:
            pltpu.sync_copy(wden.at[pl.ds(0, L)], den.at[pl.ds(dbase + doff, L)])
            doff += L
            dleft -= L
        plsc.subcore_barrier()

        mask_hi = jnp.full((L,), -65536, jnp.int32)
        cbias = c * N_pad

        def wcompute(src_v, tgt_v):
            flat_s = src_v * 2
            flat_t = tgt_v * 2 + 1
            pair_s = plsc.load_gather(alpha_v, [flat_s >> 7, flat_s & 127])
            pair_t = plsc.load_gather(alpha_v, [flat_t >> 7, flat_t & 127])
            ws = []
            for hp in range(2):
                if hp == 0:
                    a_s = plsc.bitcast(pair_s << 16, jnp.float32)
                    a_t = plsc.bitcast(pair_t << 16, jnp.float32)
                else:
                    a_s = plsc.bitcast(pair_s & mask_hi, jnp.float32)
                    a_t = plsc.bitcast(pair_t & mask_hi, jnp.float32)
                lg = a_s + a_t
                ws.append(jnp.exp(-jnp.maximum(lg, lg * 0.2)))
            return ws

        def scale32(rbuf, wv):
            for i in range(2 * L):
                half = i // L
                s0 = wv[half][0][i % L]
                s1 = wv[half][1][i % L]
                for q in range(NV):
                    sc = s0 if q < NV // 2 else s1
                    rbuf[i, pl.ds(q * L, L)] = rbuf[i, pl.ds(q * L, L)] * sc

        def blk_body(b, carry):
            pltpu.sync_copy(srch.at[s, pl.ds(b * (BLK // 8), BLK // 8)], srcb)
            pltpu.sync_copy(tgth.at[s, pl.ds(b * (BLK // 8), BLK // 8)], tgtb)

            def pair_body(k2, carry2):
                rowA = k2 >> 1
                cA = (k2 & 1) * 64
                cB = cA + 32
                svs, tvs = [], []
                for cc in (cA, cB):
                    for half in range(2):
                        svs.append(srcb[rowA, pl.ds(cc + half * L, L)])
                        tvs.append(tgtb[rowA, pl.ds(cc + half * L, L)])
                ixg0[pl.ds(0, L)] = svs[0] + cbias
                ixg0[pl.ds(L, L)] = svs[1] + cbias
                ixg1[pl.ds(0, L)] = svs[2] + cbias
                ixg1[pl.ds(L, L)] = svs[3] + cbias
                ixs0[pl.ds(0, L)] = tvs[0]
                ixs0[pl.ds(L, L)] = tvs[1]
                ixs1[pl.ds(0, L)] = tvs[2]
                ixs1[pl.ds(L, L)] = tvs[3]
                ixd0[pl.ds(0, L)] = tvs[0] >> 3
                ixd0[pl.ds(L, L)] = tvs[1] >> 3
                ixd1[pl.ds(0, L)] = tvs[2] >> 3
                ixd1[pl.ds(L, L)] = tvs[3] >> 3
                gA = pltpu.async_copy(h2.at[ixg0], rows, semg0)
                gB = pltpu.async_copy(h2.at[ixg1], rows1, semg1)
                wv = [wcompute(svs[u], tvs[u]) for u in range(4)]
                wvA = wv[:2]
                wvB = wv[2:]

                def den_fill(u0, wv2):
                    dc0 = (tvs[u0] & 7) * L
                    dc1 = (tvs[u0 + 1] & 7) * L
                    plsc.store_scatter(wden, [iota, dc0], wv2[0][0])
                    plsc.store_scatter(wden, [iota, dc0 + 1], wv2[0][1])
                    plsc.store_scatter(wden, [iota + L, dc1], wv2[1][0])
                    plsc.store_scatter(wden, [iota + L, dc1 + 1], wv2[1][1])
                    return dc0, dc1

                def den_zero(dc0, dc1):
                    plsc.store_scatter(wden, [iota, dc0], zvec)
                    plsc.store_scatter(wden, [iota, dc0 + 1], zvec)
                    plsc.store_scatter(wden, [iota + L, dc1], zvec)
                    plsc.store_scatter(wden, [iota + L, dc1 + 1], zvec)

                dcA = den_fill(0, wvA)
                sdA = pltpu.async_copy(wden, den.at[ixd0], semd0, add=True)
                gA.wait()
                scale32(rows, wvA)
                srA = pltpu.async_copy(rows, acc.at[ixs0], semsr, add=True)
                gB.wait()
                scale32(rows1, wvB)
                srB = pltpu.async_copy(rows1, acc.at[ixs1], semsr, add=True)
                sdA.wait()
                den_zero(*dcA)
                dcB = den_fill(2, wvB)
                sdB = pltpu.async_copy(wden, den.at[ixd1], semd0, add=True)
                srA.wait()
                srB.wait()
                sdB.wait()
                den_zero(*dcB)
                return carry2

            lax.fori_loop(0, BLK // 4, pair_body, 0)
            return carry

        lax.fori_loop(0, n_blk, blk_body, 0)
        plsc.subcore_barrier()

        def fin_body(k2, carry):
            gbase = zbase + k2 * 128
            pltpu.sync_copy(den.at[pl.ds((zbase >> 3) + k2 * L, L)],
                            wden.at[pl.ds(0, L)])
            for b in range(8):
                base = gbase + b * L
                pltpu.sync_copy(acc.at[pl.ds(base, L)], rows.at[pl.ds(0, L)])
                for r in range(L):
                    dv = wden[2 * b + r // 8, pl.ds((r % 8) * L, L)]
                    inv = jnp.ones((L,), jnp.float32) / jnp.maximum(dv, 1e-10)
                    d0 = inv[0]
                    d1 = inv[1]
                    for q in range(NV):
                        dd = d0 if q < NV // 2 else d1
                        rows[r, pl.ds(q * L, L)] = rows[r, pl.ds(q * L, L)] * dd
                pltpu.sync_copy(rows.at[pl.ds(0, L)], out.at[c, pl.ds(base, L)])
            return carry

        lax.fori_loop(0, rows_per_tile // 128, fin_body, 0)

    return sc_fn


@jax.jit
def kernel(x, edge_index, W, att, bias):
    N, IN = x.shape
    E = edge_index.shape[1]
    H = att.shape[1]
    O = att.shape[2] // 2
    HO = H * O
    HO2 = HO // NC
    HPC = H // NC

    N_pad = ((N + NS * L - 1) // (NS * L)) * (NS * L)
    EPT = (E + NS - 1) // NS
    CPW = ((EPT + L * BLK - 1) // (L * BLK)) * BLK
    BN = 512
    NB = N_pad // BN

    x_pad = jnp.pad(x, ((0, N_pad - N), (0, 0)))
    W2 = W.reshape(NC, HO2, IN)

    att_s = att[0, :, :O]
    att_t = att[0, :, O:]
    M = jnp.zeros((NC, HO2, L), jnp.float32)
    for c in range(NC):
        for hp in range(HPC):
            g = c * HPC + hp
            sl = slice(hp * O, (hp + 1) * O)
            M = M.at[c, sl, hp].set(att_s[g])
            M = M.at[c, sl, 2 + hp].set(att_t[g])

    h2, alpha = pl.pallas_call(
        _tc_body,
        grid=(NC, NB),
        in_specs=[
            pl.BlockSpec((BN, IN), lambda c, i: (i, 0)),
            pl.BlockSpec((1, HO2, IN), lambda c, i: (c, 0, 0)),
            pl.BlockSpec((1, HO2, L), lambda c, i: (c, 0, 0)),
        ],
        out_specs=[
            pl.BlockSpec((1, BN, HO2), lambda c, i: (c, i, 0)),
            pl.BlockSpec((1, BN, 2), lambda c, i: (c, i, 0)),
        ],
        out_shape=[
            jax.ShapeDtypeStruct((NC, N_pad, HO2), jnp.float32),
            jax.ShapeDtypeStruct((NC, N_pad, 2), jnp.int32),
        ],
    )(x_pad, W2, M)

    src = edge_index[0].astype(jnp.int32).reshape(NS, EPT)
    tgt = edge_index[1].astype(jnp.int32).reshape(NS, EPT)
    padc = CPW * L - EPT
    if padc:
        src = jnp.pad(src, ((0, 0), (0, padc)), constant_values=N)
        tgt = jnp.pad(tgt, ((0, 0), (0, padc)), constant_values=N)
    srch = src.reshape(NS, CPW * L // 128, 128)
    tgth = tgt.reshape(NS, CPW * L // 128, 128)

    alpha_flat = alpha.reshape(NC, N_pad // 64, 128)

    sc_fn = _make_sc_kernel(N_pad, CPW, HO2)
    out2 = sc_fn(h2.reshape(NC * N_pad, HO2), alpha_flat, srch, tgth)

    out = jnp.concatenate([out2[0, :N], out2[1, :N]], axis=1)
    return out + bias


# speedup vs baseline: 1.0013x; 1.0013x over previous
import functools

import jax
import jax.numpy as jnp
from jax import lax
from jax.experimental import pallas as pl
from jax.experimental.pallas import tpu as pltpu
from jax.experimental.pallas import tpu_sc as plsc

NC = 2
NS = 16
L = 16
BLK = 64


def _tc_body(x_ref, w_ref, m_ref, h_ref, a_ref):
    p = lax.dot_general(
        x_ref[...], w_ref[0],
        dimension_numbers=(((1,), (1,)), ((), ())),
        preferred_element_type=jnp.float32,
    )
    h_ref[0] = p
    al = jnp.dot(p, m_ref[0], preferred_element_type=jnp.float32)
    au = lax.bitcast_convert_type(al, jnp.uint32) + jnp.uint32(0x8000)
    lo_s = au[:, 0:1] >> 16
    hi_s = au[:, 1:2] & jnp.uint32(0xFFFF0000)
    lo_t = au[:, 2:3] >> 16
    hi_t = au[:, 3:4] & jnp.uint32(0xFFFF0000)
    packed = jnp.concatenate([lo_s | hi_s, lo_t | hi_t], axis=1)
    a_ref[0] = lax.bitcast_convert_type(packed, jnp.int32)


def _make_sc_kernel(N_pad, CPW, HO2):
    NV = HO2 // L
    mesh = plsc.VectorSubcoreMesh(core_axis_name="c", subcore_axis_name="s")
    rows_per_tile = N_pad // NS
    n_fin = rows_per_tile // L
    n_blk = CPW // BLK

    @functools.partial(
        pl.kernel,
        out_type=jax.ShapeDtypeStruct((NC, N_pad, HO2), jnp.float32),
        mesh=mesh,
        scratch_types=[
            pltpu.VMEM((N_pad // 64, 128), jnp.int32),
            pltpu.VMEM((BLK // 8, 128), jnp.int32),
            pltpu.VMEM((BLK // 8, 128), jnp.int32),
            pltpu.VMEM((2 * L,), jnp.int32),
            pltpu.VMEM((2 * L,), jnp.int32),
            pltpu.VMEM((2 * L,), jnp.int32),
            pltpu.VMEM((2 * L,), jnp.int32),
            pltpu.VMEM((2 * L, HO2), jnp.float32),
            pltpu.VMEM((2 * L, HO2), jnp.float32),
            pltpu.VMEM((L, 128), jnp.float32),
            pltpu.VMEM((L, 128), jnp.float32),
            pltpu.VMEM_SHARED((N_pad, HO2), jnp.float32),
            pltpu.VMEM_SHARED((N_pad * L // 128, 128), jnp.float32),
        ] + [pltpu.SemaphoreType.DMA] * 7,
        compiler_params=pltpu.CompilerParams(needs_layout_passes=False),
    )
    def sc_fn(h2, alpha_h, srch, tgth, out, alpha_v, srcb, tgtb,
              ixg0, ixg1, ixs0, ixs1, rows,
              rows1, wden, wden1, acc, den,
              semg0, semg1, semg2, semg3, semsr, semd0, semd1):
        c = lax.axis_index("c")
        s = lax.axis_index("s")
        zvec = jnp.zeros((L,), jnp.float32)
        iota = lax.iota(jnp.int32, L)

        pltpu.sync_copy(alpha_h.at[c], alpha_v)

        for r in range(2 * L):
            for q in range(8):
                rows[r, pl.ds(q * L, L)] = zvec
        for r in range(L):
            for q in range(8):
                wden[r, pl.ds(q * L, L)] = zvec
                wden1[r, pl.ds(q * L, L)] = zvec
        zbase = s * rows_per_tile

        def zero_body(k, carry):
            pltpu.sync_copy(rows, acc.at[pl.ds(zbase + k * 2 * L, 2 * L)])
            return carry

        lax.fori_loop(0, n_fin // 2, zero_body, 0)
        dbase = s * (rows_per_tile // 8)
        for k in range(rows_per_tile // 8 // L):
            pltpu.sync_copy(wden, den.at[pl.ds(dbase + k * L, L)])
        plsc.subcore_barrier()

        mask_hi = jnp.full((L,), -65536, jnp.int32)
        cbias = c * N_pad

        def wcompute(src_v, tgt_v):
            flat_s = src_v * 2
            flat_t = tgt_v * 2 + 1
            pair_s = plsc.load_gather(alpha_v, [flat_s >> 7, flat_s & 127])
            pair_t = plsc.load_gather(alpha_v, [flat_t >> 7, flat_t & 127])
            ws = []
            for hp in range(2):
                if hp == 0:
                    a_s = plsc.bitcast(pair_s << 16, jnp.float32)
                    a_t = plsc.bitcast(pair_t << 16, jnp.float32)
                else:
                    a_s = plsc.bitcast(pair_s & mask_hi, jnp.float32)
                    a_t = plsc.bitcast(pair_t & mask_hi, jnp.float32)
                lg = a_s + a_t
                ws.append(jnp.exp(-jnp.maximum(lg, lg * 0.2)))
            return ws

        def scale32(rbuf, wv):
            for i in range(2 * L):
                half = i // L
                s0 = wv[half][0][i % L]
                s1 = wv[half][1][i % L]
                for q in range(NV):
                    sc = s0 if q < NV // 2 else s1
                    rbuf[i, pl.ds(q * L, L)] = rbuf[i, pl.ds(q * L, L)] * sc

        def blk_body(b, carry):
            pltpu.sync_copy(srch.at[s, pl.ds(b * (BLK // 8), BLK // 8)], srcb)
            pltpu.sync_copy(tgth.at[s, pl.ds(b * (BLK // 8), BLK // 8)], tgtb)

            def pair_body(k2, carry2):
                rowA = k2 >> 1
                cA = (k2 & 1) * 64
                cB = cA + 32
                svs, tvs = [], []
                for cc in (cA, cB):
                    for half in range(2):
                        svs.append(srcb[rowA, pl.ds(cc + half * L, L)])
                        tvs.append(tgtb[rowA, pl.ds(cc + half * L, L)])
                ixg0[pl.ds(0, L)] = svs[0] + cbias
                ixg0[pl.ds(L, L)] = svs[1] + cbias
                ixg1[pl.ds(0, L)] = svs[2] + cbias
                ixg1[pl.ds(L, L)] = svs[3] + cbias
                ixs0[pl.ds(0, L)] = tvs[0]
                ixs0[pl.ds(L, L)] = tvs[1]
                ixs1[pl.ds(0, L)] = tvs[2]
                ixs1[pl.ds(L, L)] = tvs[3]
                gA = pltpu.async_copy(h2.at[ixg0], rows, semg0)
                gB = pltpu.async_copy(h2.at[ixg1], rows1, semg1)
                wv = [wcompute(svs[u], tvs[u]) for u in range(4)]
                wvA = wv[:2]
                wvB = wv[2:]
                wdb = [wden, wden1]
                dsem = [semd0, semd1]

                def den_fill(u, wv1):
                    dc = (tvs[u] & 7) * L
                    plsc.store_scatter(wdb[u & 1], [iota, dc], wv1[0])
                    plsc.store_scatter(wdb[u & 1], [iota, dc + 1], wv1[1])
                    return dc

                def den_zero(u, dc):
                    plsc.store_scatter(wdb[u & 1], [iota, dc], zvec)
                    plsc.store_scatter(wdb[u & 1], [iota, dc + 1], zvec)

                def den_send(u):
                    return pltpu.async_copy(
                        wdb[u & 1], den.at[tvs[u] >> 3], dsem[u & 1], add=True)

                dc0 = den_fill(0, wv[0])
                sd0 = den_send(0)
                dc1 = den_fill(1, wv[1])
                sd1 = den_send(1)
                gA.wait()
                scale32(rows, wvA)
                srA = pltpu.async_copy(rows, acc.at[ixs0], semsr, add=True)
                sd0.wait()
                den_zero(0, dc0)
                dc2 = den_fill(2, wv[2])
                sd2 = den_send(2)
                sd1.wait()
                den_zero(1, dc1)
                dc3 = den_fill(3, wv[3])
                sd3 = den_send(3)
                gB.wait()
                scale32(rows1, wvB)
                srB = pltpu.async_copy(rows1, acc.at[ixs1], semsr, add=True)
                sd2.wait()
                den_zero(2, dc2)
                sd3.wait()
                den_zero(3, dc3)
                srA.wait()
                srB.wait()
                return carry2

            lax.fori_loop(0, BLK // 4, pair_body, 0)
            return carry

        lax.fori_loop(0, n_blk, blk_body, 0)
        plsc.subcore_barrier()

        def fin_body(k2, carry):
            gbase = zbase + k2 * 128
            pltpu.sync_copy(den.at[pl.ds((zbase >> 3) + k2 * L, L)],
                            wden.at[pl.ds(0, L)])
            for b in range(8):
                base = gbase + b * L
                pltpu.sync_copy(acc.at[pl.ds(base, L)], rows.at[pl.ds(0, L)])
                for r in range(L):
                    dv = wden[2 * b + r // 8, pl.ds((r % 8) * L, L)]
                    inv = jnp.ones((L,), jnp.float32) / jnp.maximum(dv, 1e-10)
                    d0 = inv[0]
                    d1 = inv[1]
                    for q in range(NV):
                        dd = d0 if q < NV // 2 else d1
                        rows[r, pl.ds(q * L, L)] = rows[r, pl.ds(q * L, L)] * dd
                pltpu.sync_copy(rows.at[pl.ds(0, L)], out.at[c, pl.ds(base, L)])
            return carry

        lax.fori_loop(0, rows_per_tile // 128, fin_body, 0)

    return sc_fn


@jax.jit
def kernel(x, edge_index, W, att, bias):
    N, IN = x.shape
    E = edge_index.shape[1]
    H = att.shape[1]
    O = att.shape[2] // 2
    HO = H * O
    HO2 = HO // NC
    HPC = H // NC

    N_pad = ((N + NS * L - 1) // (NS * L)) * (NS * L)
    EPT = (E + NS - 1) // NS
    CPW = ((EPT + L * BLK - 1) // (L * BLK)) * BLK
    BN = 512
    NB = N_pad // BN

    x_pad = jnp.pad(x, ((0, N_pad - N), (0, 0)))
    W2 = W.reshape(NC, HO2, IN)

    att_s = att[0, :, :O]
    att_t = att[0, :, O:]
    M = jnp.zeros((NC, HO2, L), jnp.float32)
    for c in range(NC):
        for hp in range(HPC):
            g = c * HPC + hp
            sl = slice(hp * O, (hp + 1) * O)
            M = M.at[c, sl, hp].set(att_s[g])
            M = M.at[c, sl, 2 + hp].set(att_t[g])

    h2, alpha = pl.pallas_call(
        _tc_body,
        grid=(NC, NB),
        in_specs=[
            pl.BlockSpec((BN, IN), lambda c, i: (i, 0)),
            pl.BlockSpec((1, HO2, IN), lambda c, i: (c, 0, 0)),
            pl.BlockSpec((1, HO2, L), lambda c, i: (c, 0, 0)),
        ],
        out_specs=[
            pl.BlockSpec((1, BN, HO2), lambda c, i: (c, i, 0)),
            pl.BlockSpec((1, BN, 2), lambda c, i: (c, i, 0)),
        ],
        out_shape=[
            jax.ShapeDtypeStruct((NC, N_pad, HO2), jnp.float32),
            jax.ShapeDtypeStruct((NC, N_pad, 2), jnp.int32),
        ],
    )(x_pad, W2, M)

    src = edge_index[0].astype(jnp.int32).reshape(NS, EPT)
    tgt = edge_index[1].astype(jnp.int32).reshape(NS, EPT)
    padc = CPW * L - EPT
    if padc:
        src = jnp.pad(src, ((0, 0), (0, padc)), constant_values=N)
        tgt = jnp.pad(tgt, ((0, 0), (0, padc)), constant_values=N)
    srch = src.reshape(NS, CPW * L // 128, 128)
    tgth = tgt.reshape(NS, CPW * L // 128, 128)

    alpha_flat = alpha.reshape(NC, N_pad // 64, 128)

    sc_fn = _make_sc_kernel(N_pad, CPW, HO2)
    out2 = sc_fn(h2.reshape(NC * N_pad, HO2), alpha_flat, srch, tgth)

    out = jnp.concatenate([out2[0, :N], out2[1, :N]], axis=1)
    return out + bias
